# Initial kernel scaffold; baseline (speedup 1.0000x reference)
import jax
import jax.numpy as jnp
from jax.experimental import pallas as pl


def kernel(features, boxes, scores, labels, W1, b1, W2, b2):
    raise NotImplementedError("write your pallas kernel here")


# speedup vs baseline: 13.3450x; 13.3450x over previous
import jax
import jax.numpy as jnp
from jax.experimental import pallas as pl

HUMAN_IDX = 0
NUM_CLASSES = 117
NMS_THRESH = 0.5
MAX_HUMAN = 10
MAX_OBJECT = 10
IMG_SIZE = 1024.0
STRIDE = 16.0
POOL = 7
N_BOXES = 5000
C_FEAT = 256

NP_PAD = 5120
ROWS = 40
COLS = 128


def _nms_kernel(x1_ref, y1_ref, x2_ref, y2_ref, keep_ref):
    x1 = x1_ref[...]
    y1 = y1_ref[...]
    x2 = x2_ref[...]
    y2 = y2_ref[...]
    area = (x2 - x1) * (y2 - y1)
    gidx = (jax.lax.broadcasted_iota(jnp.int32, (ROWS, COLS), 0) * COLS
            + jax.lax.broadcasted_iota(jnp.int32, (ROWS, COLS), 1))

    def body(i, keep):
        sel = (gidx == i).astype(jnp.float32)
        x1i = jnp.sum(x1 * sel)
        y1i = jnp.sum(y1 * sel)
        x2i = jnp.sum(x2 * sel)
        y2i = jnp.sum(y2 * sel)
        keep_i = jnp.sum(keep * sel)
        area_i = (x2i - x1i) * (y2i - y1i)
        ltx = jnp.maximum(x1i, x1)
        lty = jnp.maximum(y1i, y1)
        rbx = jnp.minimum(x2i, x2)
        rby = jnp.minimum(y2i, y2)
        wx = jnp.clip(rbx - ltx, 0.0)
        wy = jnp.clip(rby - lty, 0.0)
        inter = wx * wy
        union = area_i + area - inter
        iou = inter / jnp.maximum(union, 1e-6)
        sup = ((iou > NMS_THRESH) & (gidx > i)).astype(jnp.float32) * keep_i
        return keep * (1.0 - sup)

    keep = jax.lax.fori_loop(0, N_BOXES, body, jnp.ones((ROWS, COLS), jnp.float32))
    keep_ref[...] = keep


def _head_kernel(feat_ref, box_ref, prior_ref, w1_ref, b1_ref, w2_ref, b2_ref,
                 out_ref):
    b = box_ref[...]
    x1 = b[:, 0:1] / STRIDE
    y1 = b[:, 1:2] / STRIDE
    x2 = b[:, 2:3] / STRIDE
    y2 = b[:, 3:4] / STRIDE
    p49 = jax.lax.broadcasted_iota(jnp.int32, (1, POOL * POOL), 1)
    prow = (p49 // POOL).astype(jnp.float32)
    pcol = (p49 % POOL).astype(jnp.float32)
    gy = y1 + (prow + 0.5) * (y2 - y1) / POOL
    gx = x1 + (pcol + 0.5) * (x2 - x1) / POOL
    y0 = jnp.clip(jnp.floor(gy), 0.0, 62.0)
    x0 = jnp.clip(jnp.floor(gx), 0.0, 62.0)
    wy = jnp.clip(gy - y0, 0.0, 1.0)
    wx = jnp.clip(gx - x0, 0.0, 1.0)
    pos00 = y0.astype(jnp.int32) * 64 + x0.astype(jnp.int32)
    col = jax.lax.broadcasted_iota(jnp.int32, (1, 64 * 64), 1)

    s_mat = jnp.zeros((2 * MAX_HUMAN, 64 * 64), jnp.float32)
    for p in range(POOL * POOL):
        base = pos00[:, p:p + 1]
        wyp = wy[:, p:p + 1]
        wxp = wx[:, p:p + 1]
        s_mat = s_mat + (col == base).astype(jnp.float32) * ((1 - wyp) * (1 - wxp))
        s_mat = s_mat + (col == base + 1).astype(jnp.float32) * ((1 - wyp) * wxp)
        s_mat = s_mat + (col == base + 64).astype(jnp.float32) * (wyp * (1 - wxp))
        s_mat = s_mat + (col == base + 65).astype(jnp.float32) * (wyp * wxp)

    pooled = jax.lax.dot_general(
        s_mat, feat_ref[...], (((1,), (1,)), ((), ())),
        preferred_element_type=jnp.float32) * (1.0 / (POOL * POOL))

    h_feat = pooled[0:MAX_HUMAN, :]
    o_feat = pooled[MAX_HUMAN:, :]
    w1a = w1_ref[0:C_FEAT, :]
    w1b = w1_ref[C_FEAT:, :]
    hp = jnp.dot(h_feat, w1a, preferred_element_type=jnp.float32)
    op = jnp.dot(o_feat, w1b, preferred_element_type=jnp.float32)

    pidx = jax.lax.broadcasted_iota(jnp.int32, (MAX_HUMAN * MAX_OBJECT, 1), 0)
    k10 = jax.lax.broadcasted_iota(jnp.int32, (1, MAX_HUMAN), 1)
    e_h = (pidx // MAX_OBJECT == k10).astype(jnp.float32)
    e_o = (pidx % MAX_OBJECT == k10).astype(jnp.float32)
    h_exp = jnp.dot(e_h, hp, preferred_element_type=jnp.float32)
    o_exp = jnp.dot(e_o, op, preferred_element_type=jnp.float32)
    hidden = jax.nn.relu(h_exp + o_exp + b1_ref[...])
    logits = jnp.dot(hidden, w2_ref[...], preferred_element_type=jnp.float32) + b2_ref[...]
    out_ref[...] = jax.nn.sigmoid(logits) * prior_ref[...]


def kernel(features, boxes, scores, labels, W1, b1, W2, b2):
    n = N_BOXES
    order = jnp.argsort(-scores)
    labels_f = labels.astype(jnp.float32)
    bo = boxes + labels_f[:, None] * (IMG_SIZE + 2.0)
    bs = bo[order]
    l_sorted = labels[order]

    npad = NP_PAD - n
    px1 = 1e7 + jnp.arange(npad, dtype=jnp.float32) * 100.0
    pads = jnp.stack([px1, jnp.zeros((npad,), jnp.float32),
                      px1 + 1.0, jnp.ones((npad,), jnp.float32)], axis=1)
    full = jnp.concatenate([bs, pads], axis=0)
    x1 = full[:, 0].reshape(ROWS, COLS)
    y1 = full[:, 1].reshape(ROWS, COLS)
    x2 = full[:, 2].reshape(ROWS, COLS)
    y2 = full[:, 3].reshape(ROWS, COLS)

    keep_f = pl.pallas_call(
        _nms_kernel,
        out_shape=jax.ShapeDtypeStruct((ROWS, COLS), jnp.float32),
    )(x1, y1, x2, y2)
    keep = keep_f.reshape(-1)[:n] > 0.5

    idx = jnp.arange(n)
    kept_h = keep & (l_sorted == HUMAN_IDX)
    kept_o = keep & (l_sorted != HUMAN_IDX)
    (h_pos,) = jnp.nonzero(kept_h, size=MAX_HUMAN, fill_value=n)
    (o_pos,) = jnp.nonzero(kept_o, size=MAX_OBJECT, fill_value=n)
    sel_sorted = jnp.concatenate([h_pos, o_pos])
    sel = order[sel_sorted]
    b_s = boxes[sel]
    s_s = scores[sel]
    l_s = labels[sel]
    nh = MAX_HUMAN

    no = MAX_OBJECT
    hi = jnp.arange(nh * no) // no
    oi = jnp.arange(nh * no) % no
    prior = (s_s[:nh][hi] * s_s[nh:][oi])[:, None]

    pred = pl.pallas_call(
        _head_kernel,
        out_shape=jax.ShapeDtypeStruct((nh * no, NUM_CLASSES), jnp.float32),
    )(features.reshape(C_FEAT, 64 * 64), b_s, prior,
      W1, b1.reshape(1, -1), W2, b2.reshape(1, -1))

    P = nh * no
    K = NUM_CLASSES
    x = (jnp.arange(P * K) // K).astype(jnp.int32)
    y = (jnp.arange(P * K) % K).astype(jnp.int32)
    pred_scores = pred.reshape(-1)
    boxes_h = b_s[:nh][hi]
    boxes_o = b_s[nh:][oi]
    return boxes_h, boxes_o, x, y, pred_scores, l_s[nh:][oi]


# speedup vs baseline: 46.9015x; 3.5145x over previous
import jax
import jax.numpy as jnp
from jax.experimental import pallas as pl

HUMAN_IDX = 0
NUM_CLASSES = 117
NMS_THRESH = 0.5
MAX_HUMAN = 10
MAX_OBJECT = 10
IMG_SIZE = 1024.0
STRIDE = 16.0
POOL = 7
N_BOXES = 5000
C_FEAT = 256

NP_PAD = 5120
ROWS = 40
COLS = 128


def _nms_kernel(x1_ref, y1_ref, x2_ref, y2_ref, hf_ref, of_ref, keep_ref):
    x1 = x1_ref[...]
    y1 = y1_ref[...]
    x2 = x2_ref[...]
    y2 = y2_ref[...]
    hf = hf_ref[...]
    of = of_ref[...]
    area = (x2 - x1) * (y2 - y1)
    gidx = (jax.lax.broadcasted_iota(jnp.int32, (ROWS, COLS), 0) * COLS
            + jax.lax.broadcasted_iota(jnp.int32, (ROWS, COLS), 1))

    def cond(c):
        i, nh, no, _ = c
        return (i < N_BOXES) & ((nh < MAX_HUMAN) | (no < MAX_OBJECT))

    def body(c):
        i, nh, no, keep = c
        sel = (gidx == i).astype(jnp.float32)
        x1i = jnp.sum(x1 * sel)
        y1i = jnp.sum(y1 * sel)
        x2i = jnp.sum(x2 * sel)
        y2i = jnp.sum(y2 * sel)
        keep_i = jnp.sum(keep * sel)
        nh = nh + keep_i * jnp.sum(hf * sel)
        no = no + keep_i * jnp.sum(of * sel)
        area_i = (x2i - x1i) * (y2i - y1i)
        ltx = jnp.maximum(x1i, x1)
        lty = jnp.maximum(y1i, y1)
        rbx = jnp.minimum(x2i, x2)
        rby = jnp.minimum(y2i, y2)
        wx = jnp.clip(rbx - ltx, 0.0)
        wy = jnp.clip(rby - lty, 0.0)
        inter = wx * wy
        union = area_i + area - inter
        iou = inter / jnp.maximum(union, 1e-6)
        sup = ((iou > NMS_THRESH) & (gidx > i)).astype(jnp.float32) * keep_i
        return i + 1, nh, no, keep * (1.0 - sup)

    init = (jnp.int32(0), jnp.float32(0), jnp.float32(0),
            jnp.ones((ROWS, COLS), jnp.float32))
    _, _, _, keep = jax.lax.while_loop(cond, body, init)
    keep_ref[...] = keep


def _head_kernel(feat_ref, box_ref, prior_ref, w1_ref, b1_ref, w2_ref, b2_ref,
                 out_ref):
    b = box_ref[...]
    x1 = b[:, 0:1] / STRIDE
    y1 = b[:, 1:2] / STRIDE
    x2 = b[:, 2:3] / STRIDE
    y2 = b[:, 3:4] / STRIDE
    p49 = jax.lax.broadcasted_iota(jnp.int32, (1, POOL * POOL), 1)
    prow = (p49 // POOL).astype(jnp.float32)
    pcol = (p49 % POOL).astype(jnp.float32)
    gy = y1 + (prow + 0.5) * (y2 - y1) / POOL
    gx = x1 + (pcol + 0.5) * (x2 - x1) / POOL
    y0 = jnp.clip(jnp.floor(gy), 0.0, 62.0)
    x0 = jnp.clip(jnp.floor(gx), 0.0, 62.0)
    wy = jnp.clip(gy - y0, 0.0, 1.0)
    wx = jnp.clip(gx - x0, 0.0, 1.0)
    pos00 = y0.astype(jnp.int32) * 64 + x0.astype(jnp.int32)
    col = jax.lax.broadcasted_iota(jnp.int32, (1, 64 * 64), 1)

    s_mat = jnp.zeros((2 * MAX_HUMAN, 64 * 64), jnp.float32)
    for p in range(POOL * POOL):
        base = pos00[:, p:p + 1]
        wyp = wy[:, p:p + 1]
        wxp = wx[:, p:p + 1]
        s_mat = s_mat + (col == base).astype(jnp.float32) * ((1 - wyp) * (1 - wxp))
        s_mat = s_mat + (col == base + 1).astype(jnp.float32) * ((1 - wyp) * wxp)
        s_mat = s_mat + (col == base + 64).astype(jnp.float32) * (wyp * (1 - wxp))
        s_mat = s_mat + (col == base + 65).astype(jnp.float32) * (wyp * wxp)

    pooled = jax.lax.dot_general(
        s_mat, feat_ref[...], (((1,), (1,)), ((), ())),
        preferred_element_type=jnp.float32) * (1.0 / (POOL * POOL))

    h_feat = pooled[0:MAX_HUMAN, :]
    o_feat = pooled[MAX_HUMAN:, :]
    w1a = w1_ref[0:C_FEAT, :]
    w1b = w1_ref[C_FEAT:, :]
    hp = jnp.dot(h_feat, w1a, preferred_element_type=jnp.float32)
    op = jnp.dot(o_feat, w1b, preferred_element_type=jnp.float32)

    pidx = jax.lax.broadcasted_iota(jnp.int32, (MAX_HUMAN * MAX_OBJECT, 1), 0)
    k10 = jax.lax.broadcasted_iota(jnp.int32, (1, MAX_HUMAN), 1)
    e_h = (pidx // MAX_OBJECT == k10).astype(jnp.float32)
    e_o = (pidx % MAX_OBJECT == k10).astype(jnp.float32)
    h_exp = jnp.dot(e_h, hp, preferred_element_type=jnp.float32)
    o_exp = jnp.dot(e_o, op, preferred_element_type=jnp.float32)
    hidden = jax.nn.relu(h_exp + o_exp + b1_ref[...])
    logits = jnp.dot(hidden, w2_ref[...], preferred_element_type=jnp.float32) + b2_ref[...]
    out_ref[...] = jax.nn.sigmoid(logits) * prior_ref[...]


def kernel(features, boxes, scores, labels, W1, b1, W2, b2):
    n = N_BOXES
    order = jnp.argsort(-scores)
    labels_f = labels.astype(jnp.float32)
    bo = boxes + labels_f[:, None] * (IMG_SIZE + 2.0)
    bs = bo[order]
    l_sorted = labels[order]

    npad = NP_PAD - n
    px1 = 1e7 + jnp.arange(npad, dtype=jnp.float32) * 100.0
    pads = jnp.stack([px1, jnp.zeros((npad,), jnp.float32),
                      px1 + 1.0, jnp.ones((npad,), jnp.float32)], axis=1)
    full = jnp.concatenate([bs, pads], axis=0)
    x1 = full[:, 0].reshape(ROWS, COLS)
    y1 = full[:, 1].reshape(ROWS, COLS)
    x2 = full[:, 2].reshape(ROWS, COLS)
    y2 = full[:, 3].reshape(ROWS, COLS)
    padf = jnp.zeros((npad,), jnp.float32)
    hf = jnp.concatenate([(l_sorted == HUMAN_IDX).astype(jnp.float32), padf]
                         ).reshape(ROWS, COLS)
    of = jnp.concatenate([(l_sorted != HUMAN_IDX).astype(jnp.float32), padf]
                         ).reshape(ROWS, COLS)

    keep_f = pl.pallas_call(
        _nms_kernel,
        out_shape=jax.ShapeDtypeStruct((ROWS, COLS), jnp.float32),
    )(x1, y1, x2, y2, hf, of)
    keep = keep_f.reshape(-1)[:n] > 0.5

    idx = jnp.arange(n)
    kept_h = keep & (l_sorted == HUMAN_IDX)
    kept_o = keep & (l_sorted != HUMAN_IDX)
    (h_pos,) = jnp.nonzero(kept_h, size=MAX_HUMAN, fill_value=n)
    (o_pos,) = jnp.nonzero(kept_o, size=MAX_OBJECT, fill_value=n)
    sel_sorted = jnp.concatenate([h_pos, o_pos])
    sel = order[sel_sorted]
    b_s = boxes[sel]
    s_s = scores[sel]
    l_s = labels[sel]
    nh = MAX_HUMAN

    no = MAX_OBJECT
    hi = jnp.arange(nh * no) // no
    oi = jnp.arange(nh * no) % no
    prior = (s_s[:nh][hi] * s_s[nh:][oi])[:, None]

    pred = pl.pallas_call(
        _head_kernel,
        out_shape=jax.ShapeDtypeStruct((nh * no, NUM_CLASSES), jnp.float32),
    )(features.reshape(C_FEAT, 64 * 64), b_s, prior,
      W1, b1.reshape(1, -1), W2, b2.reshape(1, -1))

    P = nh * no
    K = NUM_CLASSES
    x = (jnp.arange(P * K) // K).astype(jnp.int32)
    y = (jnp.arange(P * K) % K).astype(jnp.int32)
    pred_scores = pred.reshape(-1)
    boxes_h = b_s[:nh][hi]
    boxes_o = b_s[nh:][oi]
    return boxes_h, boxes_o, x, y, pred_scores, l_s[nh:][oi]


# speedup vs baseline: 47.0252x; 1.0026x over previous
import functools

import jax
import jax.numpy as jnp
from jax.experimental import pallas as pl
from jax.experimental.pallas import tpu as pltpu
from jax.experimental.pallas import tpu_sc as plsc

HUMAN_IDX = 0
NUM_CLASSES = 117
NMS_THRESH = 0.5
MAX_HUMAN = 10
MAX_OBJECT = 10
IMG_SIZE = 1024.0
STRIDE = 16.0
POOL = 7
N_BOXES = 5000
C_FEAT = 256

NP_PAD = 5120
ROWS = 40
COLS = 128


TD = 128


def _make_sc_order_gather():
    info = plsc.get_sparse_core_info()
    nw = info.num_cores * info.num_subcores
    b_per_w = NP_PAD // nw
    mesh = plsc.VectorSubcoreMesh(core_axis_name="c", subcore_axis_name="s")

    @functools.partial(
        pl.kernel, mesh=mesh,
        out_type=jax.ShapeDtypeStruct((NP_PAD, TD), jnp.float32),
        scratch_types=[
            pltpu.VMEM((b_per_w,), jnp.int32),
            pltpu.VMEM((b_per_w, TD), jnp.float32),
            pltpu.SemaphoreType.DMA,
        ],
    )
    def sc_gather(table_hbm, idx_hbm, out_hbm, idx_v, rows_v, sem):
        wid = (jax.lax.axis_index("s") * info.num_cores
               + jax.lax.axis_index("c"))
        base = wid * b_per_w
        pltpu.sync_copy(idx_hbm.at[pl.ds(base, b_per_w)], idx_v)
        pltpu.async_copy(table_hbm.at[idx_v], rows_v, sem).wait()
        pltpu.sync_copy(rows_v, out_hbm.at[pl.ds(base, b_per_w)])

    return sc_gather


_sc_order_gather = _make_sc_order_gather()


def _nms_kernel(x1_ref, y1_ref, x2_ref, y2_ref, hf_ref, of_ref, keep_ref):
    x1 = x1_ref[...]
    y1 = y1_ref[...]
    x2 = x2_ref[...]
    y2 = y2_ref[...]
    hf = hf_ref[...]
    of = of_ref[...]
    area = (x2 - x1) * (y2 - y1)
    gidx = (jax.lax.broadcasted_iota(jnp.int32, (ROWS, COLS), 0) * COLS
            + jax.lax.broadcasted_iota(jnp.int32, (ROWS, COLS), 1))

    def cond(c):
        i, nh, no, _ = c
        return (i < N_BOXES) & ((nh < MAX_HUMAN) | (no < MAX_OBJECT))

    def body(c):
        i, nh, no, keep = c
        sel = (gidx == i).astype(jnp.float32)
        x1i = jnp.sum(x1 * sel)
        y1i = jnp.sum(y1 * sel)
        x2i = jnp.sum(x2 * sel)
        y2i = jnp.sum(y2 * sel)
        keep_i = jnp.sum(keep * sel)
        nh = nh + keep_i * jnp.sum(hf * sel)
        no = no + keep_i * jnp.sum(of * sel)
        area_i = (x2i - x1i) * (y2i - y1i)
        ltx = jnp.maximum(x1i, x1)
        lty = jnp.maximum(y1i, y1)
        rbx = jnp.minimum(x2i, x2)
        rby = jnp.minimum(y2i, y2)
        wx = jnp.clip(rbx - ltx, 0.0)
        wy = jnp.clip(rby - lty, 0.0)
        inter = wx * wy
        union = area_i + area - inter
        iou = inter / jnp.maximum(union, 1e-6)
        sup = ((iou > NMS_THRESH) & (gidx > i)).astype(jnp.float32) * keep_i
        return i + 1, nh, no, keep * (1.0 - sup)

    init = (jnp.int32(0), jnp.float32(0), jnp.float32(0),
            jnp.ones((ROWS, COLS), jnp.float32))
    _, _, _, keep = jax.lax.while_loop(cond, body, init)
    keep_ref[...] = keep


def _head_kernel(feat_ref, box_ref, prior_ref, w1_ref, b1_ref, w2_ref, b2_ref,
                 out_ref):
    b = box_ref[...]
    x1 = b[:, 0:1] / STRIDE
    y1 = b[:, 1:2] / STRIDE
    x2 = b[:, 2:3] / STRIDE
    y2 = b[:, 3:4] / STRIDE
    p49 = jax.lax.broadcasted_iota(jnp.int32, (1, POOL * POOL), 1)
    prow = (p49 // POOL).astype(jnp.float32)
    pcol = (p49 % POOL).astype(jnp.float32)
    gy = y1 + (prow + 0.5) * (y2 - y1) / POOL
    gx = x1 + (pcol + 0.5) * (x2 - x1) / POOL
    y0 = jnp.clip(jnp.floor(gy), 0.0, 62.0)
    x0 = jnp.clip(jnp.floor(gx), 0.0, 62.0)
    wy = jnp.clip(gy - y0, 0.0, 1.0)
    wx = jnp.clip(gx - x0, 0.0, 1.0)
    pos00 = y0.astype(jnp.int32) * 64 + x0.astype(jnp.int32)
    col = jax.lax.broadcasted_iota(jnp.int32, (1, 64 * 64), 1)

    s_mat = jnp.zeros((2 * MAX_HUMAN, 64 * 64), jnp.float32)
    for p in range(POOL * POOL):
        base = pos00[:, p:p + 1]
        wyp = wy[:, p:p + 1]
        wxp = wx[:, p:p + 1]
        s_mat = s_mat + (col == base).astype(jnp.float32) * ((1 - wyp) * (1 - wxp))
        s_mat = s_mat + (col == base + 1).astype(jnp.float32) * ((1 - wyp) * wxp)
        s_mat = s_mat + (col == base + 64).astype(jnp.float32) * (wyp * (1 - wxp))
        s_mat = s_mat + (col == base + 65).astype(jnp.float32) * (wyp * wxp)

    pooled = jax.lax.dot_general(
        s_mat, feat_ref[...], (((1,), (1,)), ((), ())),
        preferred_element_type=jnp.float32) * (1.0 / (POOL * POOL))

    h_feat = pooled[0:MAX_HUMAN, :]
    o_feat = pooled[MAX_HUMAN:, :]
    w1a = w1_ref[0:C_FEAT, :]
    w1b = w1_ref[C_FEAT:, :]
    hp = jnp.dot(h_feat, w1a, preferred_element_type=jnp.float32)
    op = jnp.dot(o_feat, w1b, preferred_element_type=jnp.float32)

    pidx = jax.lax.broadcasted_iota(jnp.int32, (MAX_HUMAN * MAX_OBJECT, 1), 0)
    k10 = jax.lax.broadcasted_iota(jnp.int32, (1, MAX_HUMAN), 1)
    e_h = (pidx // MAX_OBJECT == k10).astype(jnp.float32)
    e_o = (pidx % MAX_OBJECT == k10).astype(jnp.float32)
    h_exp = jnp.dot(e_h, hp, preferred_element_type=jnp.float32)
    o_exp = jnp.dot(e_o, op, preferred_element_type=jnp.float32)
    hidden = jax.nn.relu(h_exp + o_exp + b1_ref[...])
    logits = jnp.dot(hidden, w2_ref[...], preferred_element_type=jnp.float32) + b2_ref[...]
    out_ref[...] = jax.nn.sigmoid(logits) * prior_ref[...]


def kernel(features, boxes, scores, labels, W1, b1, W2, b2):
    n = N_BOXES
    order = jnp.argsort(-scores)
    labels_f = labels.astype(jnp.float32)
    bo = boxes + labels_f[:, None] * (IMG_SIZE + 2.0)

    npad = NP_PAD - n
    px1 = 1e7 + jnp.arange(npad, dtype=jnp.float32) * 100.0
    pads = jnp.stack([px1, jnp.zeros((npad,), jnp.float32),
                      px1 + 1.0, jnp.ones((npad,), jnp.float32)], axis=1)
    hf_col = (labels == HUMAN_IDX).astype(jnp.float32)[:, None]
    table = jnp.concatenate([
        jnp.concatenate([bo, hf_col, 1.0 - hf_col,
                         jnp.zeros((n, TD - 6), jnp.float32)], axis=1),
        jnp.concatenate([pads, jnp.zeros((npad, TD - 4), jnp.float32)], axis=1),
    ], axis=0)
    idx_pad = jnp.concatenate([order.astype(jnp.int32),
                               jnp.arange(n, NP_PAD, dtype=jnp.int32)])
    g = _sc_order_gather(table, idx_pad)
    x1 = g[:, 0].reshape(ROWS, COLS)
    y1 = g[:, 1].reshape(ROWS, COLS)
    x2 = g[:, 2].reshape(ROWS, COLS)
    y2 = g[:, 3].reshape(ROWS, COLS)
    hf = g[:, 4].reshape(ROWS, COLS)
    of = g[:, 5].reshape(ROWS, COLS)

    keep_f = pl.pallas_call(
        _nms_kernel,
        out_shape=jax.ShapeDtypeStruct((ROWS, COLS), jnp.float32),
    )(x1, y1, x2, y2, hf, of)
    keep = keep_f.reshape(-1)[:n] > 0.5

    kept_h = keep & (g[:n, 4] > 0.5)
    kept_o = keep & (g[:n, 5] > 0.5)
    (h_pos,) = jnp.nonzero(kept_h, size=MAX_HUMAN, fill_value=n)
    (o_pos,) = jnp.nonzero(kept_o, size=MAX_OBJECT, fill_value=n)
    sel_sorted = jnp.concatenate([h_pos, o_pos])
    sel = order[sel_sorted]
    b_s = boxes[sel]
    s_s = scores[sel]
    l_s = labels[sel]
    nh = MAX_HUMAN

    no = MAX_OBJECT
    hi = jnp.arange(nh * no) // no
    oi = jnp.arange(nh * no) % no
    prior = (s_s[:nh][hi] * s_s[nh:][oi])[:, None]

    pred = pl.pallas_call(
        _head_kernel,
        out_shape=jax.ShapeDtypeStruct((nh * no, NUM_CLASSES), jnp.float32),
    )(features.reshape(C_FEAT, 64 * 64), b_s, prior,
      W1, b1.reshape(1, -1), W2, b2.reshape(1, -1))

    P = nh * no
    K = NUM_CLASSES
    x = (jnp.arange(P * K) // K).astype(jnp.int32)
    y = (jnp.arange(P * K) % K).astype(jnp.int32)
    pred_scores = pred.reshape(-1)
    boxes_h = b_s[:nh][hi]
    boxes_o = b_s[nh:][oi]
    return boxes_h, boxes_o, x, y, pred_scores, l_s[nh:][oi]


# speedup vs baseline: 57.7444x; 1.2279x over previous
import functools

import jax
import jax.numpy as jnp
from jax.experimental import pallas as pl
from jax.experimental.pallas import tpu as pltpu
from jax.experimental.pallas import tpu_sc as plsc

HUMAN_IDX = 0
NUM_CLASSES = 117
NMS_THRESH = 0.5
MAX_HUMAN = 10
MAX_OBJECT = 10
IMG_SIZE = 1024.0
STRIDE = 16.0
POOL = 7
N_BOXES = 5000
C_FEAT = 256

NP_PAD = 5120
ROWS = 40
COLS = 128


TD = 128


def _make_sc_order_gather():
    info = plsc.get_sparse_core_info()
    nw = info.num_cores * info.num_subcores
    b_per_w = NP_PAD // nw
    mesh = plsc.VectorSubcoreMesh(core_axis_name="c", subcore_axis_name="s")

    @functools.partial(
        pl.kernel, mesh=mesh,
        out_type=jax.ShapeDtypeStruct((NP_PAD, TD), jnp.float32),
        scratch_types=[
            pltpu.VMEM((b_per_w,), jnp.int32),
            pltpu.VMEM((b_per_w, TD), jnp.float32),
            pltpu.SemaphoreType.DMA,
        ],
    )
    def sc_gather(table_hbm, idx_hbm, out_hbm, idx_v, rows_v, sem):
        wid = (jax.lax.axis_index("s") * info.num_cores
               + jax.lax.axis_index("c"))
        base = wid * b_per_w
        pltpu.sync_copy(idx_hbm.at[pl.ds(base, b_per_w)], idx_v)
        pltpu.async_copy(table_hbm.at[idx_v], rows_v, sem).wait()
        pltpu.sync_copy(rows_v, out_hbm.at[pl.ds(base, b_per_w)])

    return sc_gather


_SC_CACHE = {}


def _sc_order_gather(table, idx):
    if "g" not in _SC_CACHE:
        _SC_CACHE["g"] = _make_sc_order_gather()
    return _SC_CACHE["g"](table, idx)


def _nms_kernel(x1_ref, y1_ref, x2_ref, y2_ref, nht_ref, keep_ref):
    x1 = x1_ref[...]
    y1 = y1_ref[...]
    x2 = x2_ref[...]
    y2 = y2_ref[...]
    nhtot = nht_ref[0, 0]
    area = (x2 - x1) * (y2 - y1)
    gidx = (jax.lax.broadcasted_iota(jnp.int32, (ROWS, COLS), 0) * COLS
            + jax.lax.broadcasted_iota(jnp.int32, (ROWS, COLS), 1))

    def body_at(i, keep):
        sel = (gidx == i).astype(jnp.float32)
        x1i = jnp.sum(x1 * sel)
        y1i = jnp.sum(y1 * sel)
        x2i = jnp.sum(x2 * sel)
        y2i = jnp.sum(y2 * sel)
        keep_i = jnp.sum(keep * sel)
        area_i = (x2i - x1i) * (y2i - y1i)
        ltx = jnp.maximum(x1i, x1)
        lty = jnp.maximum(y1i, y1)
        rbx = jnp.minimum(x2i, x2)
        rby = jnp.minimum(y2i, y2)
        wx = jnp.clip(rbx - ltx, 0.0)
        wy = jnp.clip(rby - lty, 0.0)
        inter = wx * wy
        union = area_i + area - inter
        iou = inter / jnp.maximum(union, 1e-6)
        sup = ((iou > NMS_THRESH) & (gidx > i)).astype(jnp.float32) * keep_i
        return keep_i, keep * (1.0 - sup)

    def scan_segment(start, end, keep):
        def cond(c):
            i, cnt, _ = c
            return (i < end) & (cnt < MAX_HUMAN)

        def body(c):
            i, cnt, keep = c
            keep_i, keep = body_at(i, keep)
            return i + 1, cnt + keep_i, keep

        _, _, keep = jax.lax.while_loop(
            cond, body, (start, jnp.float32(0), keep))
        return keep

    keep = jnp.ones((ROWS, COLS), jnp.float32)
    keep = scan_segment(jnp.int32(0), nhtot, keep)
    keep = scan_segment(nhtot, jnp.int32(N_BOXES), keep)
    keep_ref[...] = keep


def _head_kernel(feat_ref, box_ref, prior_ref, w1_ref, b1_ref, w2_ref, b2_ref,
                 out_ref):
    b = box_ref[...]
    x1 = b[:, 0:1] / STRIDE
    y1 = b[:, 1:2] / STRIDE
    x2 = b[:, 2:3] / STRIDE
    y2 = b[:, 3:4] / STRIDE
    p49 = jax.lax.broadcasted_iota(jnp.int32, (1, POOL * POOL), 1)
    prow = (p49 // POOL).astype(jnp.float32)
    pcol = (p49 % POOL).astype(jnp.float32)
    gy = y1 + (prow + 0.5) * (y2 - y1) / POOL
    gx = x1 + (pcol + 0.5) * (x2 - x1) / POOL
    y0 = jnp.clip(jnp.floor(gy), 0.0, 62.0)
    x0 = jnp.clip(jnp.floor(gx), 0.0, 62.0)
    wy = jnp.clip(gy - y0, 0.0, 1.0)
    wx = jnp.clip(gx - x0, 0.0, 1.0)
    pos00 = y0.astype(jnp.int32) * 64 + x0.astype(jnp.int32)
    col = jax.lax.broadcasted_iota(jnp.int32, (1, 64 * 64), 1)

    s_mat = jnp.zeros((2 * MAX_HUMAN, 64 * 64), jnp.float32)
    for p in range(POOL * POOL):
        base = pos00[:, p:p + 1]
        wyp = wy[:, p:p + 1]
        wxp = wx[:, p:p + 1]
        s_mat = s_mat + (col == base).astype(jnp.float32) * ((1 - wyp) * (1 - wxp))
        s_mat = s_mat + (col == base + 1).astype(jnp.float32) * ((1 - wyp) * wxp)
        s_mat = s_mat + (col == base + 64).astype(jnp.float32) * (wyp * (1 - wxp))
        s_mat = s_mat + (col == base + 65).astype(jnp.float32) * (wyp * wxp)

    pooled = jax.lax.dot_general(
        s_mat, feat_ref[...], (((1,), (1,)), ((), ())),
        preferred_element_type=jnp.float32) * (1.0 / (POOL * POOL))

    h_feat = pooled[0:MAX_HUMAN, :]
    o_feat = pooled[MAX_HUMAN:, :]
    w1a = w1_ref[0:C_FEAT, :]
    w1b = w1_ref[C_FEAT:, :]
    hp = jnp.dot(h_feat, w1a, preferred_element_type=jnp.float32)
    op = jnp.dot(o_feat, w1b, preferred_element_type=jnp.float32)

    pidx = jax.lax.broadcasted_iota(jnp.int32, (MAX_HUMAN * MAX_OBJECT, 1), 0)
    k10 = jax.lax.broadcasted_iota(jnp.int32, (1, MAX_HUMAN), 1)
    e_h = (pidx // MAX_OBJECT == k10).astype(jnp.float32)
    e_o = (pidx % MAX_OBJECT == k10).astype(jnp.float32)
    h_exp = jnp.dot(e_h, hp, preferred_element_type=jnp.float32)
    o_exp = jnp.dot(e_o, op, preferred_element_type=jnp.float32)
    hidden = jax.nn.relu(h_exp + o_exp + b1_ref[...])
    logits = jnp.dot(hidden, w2_ref[...], preferred_element_type=jnp.float32) + b2_ref[...]
    out_ref[...] = jax.nn.sigmoid(logits) * prior_ref[...]


def kernel(features, boxes, scores, labels, W1, b1, W2, b2):
    n = N_BOXES
    order = jnp.argsort(-scores)
    labels_f = labels.astype(jnp.float32)
    bo = boxes + labels_f[:, None] * (IMG_SIZE + 2.0)

    npad = NP_PAD - n
    px1 = 1e7 + jnp.arange(npad, dtype=jnp.float32) * 100.0
    pads = jnp.stack([px1, jnp.zeros((npad,), jnp.float32),
                      px1 + 1.0, jnp.ones((npad,), jnp.float32)], axis=1)
    hf_col = (labels == HUMAN_IDX).astype(jnp.float32)[:, None]
    table = jnp.concatenate([
        jnp.concatenate([bo, hf_col, 1.0 - hf_col,
                         jnp.zeros((n, TD - 6), jnp.float32)], axis=1),
        jnp.concatenate([pads, jnp.zeros((npad, TD - 4), jnp.float32)], axis=1),
    ], axis=0)
    hmask_s = labels[order] == HUMAN_IDX
    part = jnp.argsort(jnp.where(hmask_s, 0, 1), stable=True).astype(jnp.int32)
    nhtot = jnp.sum(hmask_s.astype(jnp.int32))
    order_part = order.astype(jnp.int32)[part]
    idx_pad = jnp.concatenate([order_part,
                               jnp.arange(n, NP_PAD, dtype=jnp.int32)])
    g = _sc_order_gather(table, idx_pad)
    x1 = g[:, 0].reshape(ROWS, COLS)
    y1 = g[:, 1].reshape(ROWS, COLS)
    x2 = g[:, 2].reshape(ROWS, COLS)
    y2 = g[:, 3].reshape(ROWS, COLS)

    keep_f = pl.pallas_call(
        _nms_kernel,
        out_shape=jax.ShapeDtypeStruct((ROWS, COLS), jnp.float32),
    )(x1, y1, x2, y2, nhtot.reshape(1, 1))
    keep = keep_f.reshape(-1)[:n] > 0.5

    pidx = jnp.arange(n)
    kept_h = keep & (pidx < nhtot)
    kept_o = keep & (pidx >= nhtot)
    (h_pos,) = jnp.nonzero(kept_h, size=MAX_HUMAN, fill_value=n)
    (o_pos,) = jnp.nonzero(kept_o, size=MAX_OBJECT, fill_value=n)
    pos_part = jnp.concatenate([h_pos, o_pos])
    sel_sorted = jnp.where(pos_part >= n, n,
                           part[jnp.minimum(pos_part, n - 1)])
    sel = order[sel_sorted]
    b_s = boxes[sel]
    s_s = scores[sel]
    l_s = labels[sel]
    nh = MAX_HUMAN

    no = MAX_OBJECT
    hi = jnp.arange(nh * no) // no
    oi = jnp.arange(nh * no) % no
    prior = (s_s[:nh][hi] * s_s[nh:][oi])[:, None]

    pred = pl.pallas_call(
        _head_kernel,
        out_shape=jax.ShapeDtypeStruct((nh * no, NUM_CLASSES), jnp.float32),
    )(features.reshape(C_FEAT, 64 * 64), b_s, prior,
      W1, b1.reshape(1, -1), W2, b2.reshape(1, -1))

    P = nh * no
    K = NUM_CLASSES
    x = (jnp.arange(P * K) // K).astype(jnp.int32)
    y = (jnp.arange(P * K) % K).astype(jnp.int32)
    pred_scores = pred.reshape(-1)
    boxes_h = b_s[:nh][hi]
    boxes_o = b_s[nh:][oi]
    return boxes_h, boxes_o, x, y, pred_scores, l_s[nh:][oi]


# speedup vs baseline: 61.1994x; 1.0598x over previous
import functools

import jax
import jax.numpy as jnp
from jax.experimental import pallas as pl
from jax.experimental.pallas import tpu as pltpu
from jax.experimental.pallas import tpu_sc as plsc

HUMAN_IDX = 0
NUM_CLASSES = 117
NMS_THRESH = 0.5
MAX_HUMAN = 10
MAX_OBJECT = 10
IMG_SIZE = 1024.0
STRIDE = 16.0
POOL = 7
N_BOXES = 5000
C_FEAT = 256

NP_PAD = 5120
ROWS = 40
COLS = 128


TD = 128


def _make_sc_order_gather():
    info = plsc.get_sparse_core_info()
    nw = info.num_cores * info.num_subcores
    b_per_w = NP_PAD // nw
    mesh = plsc.VectorSubcoreMesh(core_axis_name="c", subcore_axis_name="s")

    @functools.partial(
        pl.kernel, mesh=mesh,
        out_type=jax.ShapeDtypeStruct((NP_PAD, TD), jnp.float32),
        scratch_types=[
            pltpu.VMEM((b_per_w,), jnp.int32),
            pltpu.VMEM((b_per_w, TD), jnp.float32),
            pltpu.SemaphoreType.DMA,
        ],
    )
    def sc_gather(table_hbm, idx_hbm, out_hbm, idx_v, rows_v, sem):
        wid = (jax.lax.axis_index("s") * info.num_cores
               + jax.lax.axis_index("c"))
        base = wid * b_per_w
        pltpu.sync_copy(idx_hbm.at[pl.ds(base, b_per_w)], idx_v)
        pltpu.async_copy(table_hbm.at[idx_v], rows_v, sem).wait()
        pltpu.sync_copy(rows_v, out_hbm.at[pl.ds(base, b_per_w)])

    return sc_gather


_SC_CACHE = {}


def _sc_order_gather(table, idx):
    if "g" not in _SC_CACHE:
        _SC_CACHE["g"] = _make_sc_order_gather()
    return _SC_CACHE["g"](table, idx)


def _nms_kernel(x1_ref, y1_ref, x2_ref, y2_ref, nht_ref, keep_ref):
    x1 = x1_ref[...]
    y1 = y1_ref[...]
    x2 = x2_ref[...]
    y2 = y2_ref[...]
    nhtot = nht_ref[0, 0]
    area = (x2 - x1) * (y2 - y1)
    gidx = (jax.lax.broadcasted_iota(jnp.int32, (ROWS, COLS), 0) * COLS
            + jax.lax.broadcasted_iota(jnp.int32, (ROWS, COLS), 1))

    def body_at(i, keep):
        sel = (gidx == i).astype(jnp.float32)
        x1i = jnp.sum(x1 * sel)
        y1i = jnp.sum(y1 * sel)
        x2i = jnp.sum(x2 * sel)
        y2i = jnp.sum(y2 * sel)
        keep_i = jnp.sum(keep * sel)
        area_i = (x2i - x1i) * (y2i - y1i)
        ltx = jnp.maximum(x1i, x1)
        lty = jnp.maximum(y1i, y1)
        rbx = jnp.minimum(x2i, x2)
        rby = jnp.minimum(y2i, y2)
        wx = jnp.clip(rbx - ltx, 0.0)
        wy = jnp.clip(rby - lty, 0.0)
        inter = wx * wy
        union = area_i + area - inter
        iou = inter / jnp.maximum(union, 1e-6)
        sup = ((iou > NMS_THRESH) & (gidx > i)).astype(jnp.float32) * keep_i
        return keep_i, keep * (1.0 - sup)

    def scan_segment(start, end, keep):
        def cond(c):
            i, cnt, _ = c
            return (i < end) & (cnt < MAX_HUMAN)

        def body(c):
            i, cnt, keep = c
            keep_i, keep = body_at(i, keep)
            return i + 1, cnt + keep_i, keep

        _, _, keep = jax.lax.while_loop(
            cond, body, (start, jnp.float32(0), keep))
        return keep

    keep = jnp.ones((ROWS, COLS), jnp.float32)
    keep = scan_segment(jnp.int32(0), nhtot, keep)
    keep = scan_segment(nhtot, jnp.int32(N_BOXES), keep)
    keep_ref[...] = keep


def _head_kernel(feat_ref, box_ref, prior_ref, w1_ref, b1_ref, w2_ref, b2_ref,
                 out_ref):
    b = box_ref[...]
    x1 = b[:, 0:1] / STRIDE
    y1 = b[:, 1:2] / STRIDE
    x2 = b[:, 2:3] / STRIDE
    y2 = b[:, 3:4] / STRIDE
    p49 = jax.lax.broadcasted_iota(jnp.int32, (1, POOL * POOL), 1)
    prow = (p49 // POOL).astype(jnp.float32)
    pcol = (p49 % POOL).astype(jnp.float32)
    gy = y1 + (prow + 0.5) * (y2 - y1) / POOL
    gx = x1 + (pcol + 0.5) * (x2 - x1) / POOL
    y0 = jnp.clip(jnp.floor(gy), 0.0, 62.0)
    x0 = jnp.clip(jnp.floor(gx), 0.0, 62.0)
    wy = jnp.clip(gy - y0, 0.0, 1.0)
    wx = jnp.clip(gx - x0, 0.0, 1.0)
    pos00 = y0.astype(jnp.int32) * 64 + x0.astype(jnp.int32)
    col = jax.lax.broadcasted_iota(jnp.int32, (1, 64 * 64), 1)

    s_mat = jnp.zeros((2 * MAX_HUMAN, 64 * 64), jnp.float32)
    for p in range(POOL * POOL):
        base = pos00[:, p:p + 1]
        wyp = wy[:, p:p + 1]
        wxp = wx[:, p:p + 1]
        s_mat = s_mat + (col == base).astype(jnp.float32) * ((1 - wyp) * (1 - wxp))
        s_mat = s_mat + (col == base + 1).astype(jnp.float32) * ((1 - wyp) * wxp)
        s_mat = s_mat + (col == base + 64).astype(jnp.float32) * (wyp * (1 - wxp))
        s_mat = s_mat + (col == base + 65).astype(jnp.float32) * (wyp * wxp)

    pooled = jax.lax.dot_general(
        s_mat, feat_ref[...], (((1,), (1,)), ((), ())),
        preferred_element_type=jnp.float32) * (1.0 / (POOL * POOL))

    h_feat = pooled[0:MAX_HUMAN, :]
    o_feat = pooled[MAX_HUMAN:, :]
    w1a = w1_ref[0:C_FEAT, :]
    w1b = w1_ref[C_FEAT:, :]
    hp = jnp.dot(h_feat, w1a, preferred_element_type=jnp.float32)
    op = jnp.dot(o_feat, w1b, preferred_element_type=jnp.float32)

    pidx = jax.lax.broadcasted_iota(jnp.int32, (MAX_HUMAN * MAX_OBJECT, 1), 0)
    k10 = jax.lax.broadcasted_iota(jnp.int32, (1, MAX_HUMAN), 1)
    e_h = (pidx // MAX_OBJECT == k10).astype(jnp.float32)
    e_o = (pidx % MAX_OBJECT == k10).astype(jnp.float32)
    h_exp = jnp.dot(e_h, hp, preferred_element_type=jnp.float32)
    o_exp = jnp.dot(e_o, op, preferred_element_type=jnp.float32)
    hidden = jax.nn.relu(h_exp + o_exp + b1_ref[...])
    logits = jnp.dot(hidden, w2_ref[...], preferred_element_type=jnp.float32) + b2_ref[...]
    out_ref[...] = jax.nn.sigmoid(logits) * prior_ref[...]


def kernel(features, boxes, scores, labels, W1, b1, W2, b2):
    n = N_BOXES
    labels_f = labels.astype(jnp.float32)
    bo = boxes + labels_f[:, None] * (IMG_SIZE + 2.0)

    npad = NP_PAD - n
    px1 = 1e7 + jnp.arange(npad, dtype=jnp.float32) * 100.0
    pads = jnp.stack([px1, jnp.zeros((npad,), jnp.float32),
                      px1 + 1.0, jnp.ones((npad,), jnp.float32)], axis=1)
    hf_col = (labels == HUMAN_IDX).astype(jnp.float32)[:, None]
    table = jnp.concatenate([
        jnp.concatenate([bo, hf_col, 1.0 - hf_col,
                         jnp.zeros((n, TD - 6), jnp.float32)], axis=1),
        jnp.concatenate([pads, jnp.zeros((npad, TD - 4), jnp.float32)], axis=1),
    ], axis=0)
    hmask = labels == HUMAN_IDX
    okey = jnp.where(hmask, -scores, 2.0 - scores)
    order_part = jnp.argsort(okey).astype(jnp.int32)
    nhtot = jnp.sum(hmask.astype(jnp.int32))
    idx_pad = jnp.concatenate([order_part,
                               jnp.arange(n, NP_PAD, dtype=jnp.int32)])
    g = _sc_order_gather(table, idx_pad)
    x1 = g[:, 0].reshape(ROWS, COLS)
    y1 = g[:, 1].reshape(ROWS, COLS)
    x2 = g[:, 2].reshape(ROWS, COLS)
    y2 = g[:, 3].reshape(ROWS, COLS)

    keep_f = pl.pallas_call(
        _nms_kernel,
        out_shape=jax.ShapeDtypeStruct((ROWS, COLS), jnp.float32),
    )(x1, y1, x2, y2, nhtot.reshape(1, 1))
    keep = keep_f.reshape(-1)[:n] > 0.5

    pidx = jnp.arange(n)
    kept_h = keep & (pidx < nhtot)
    kept_o = keep & (pidx >= nhtot)
    (h_pos,) = jnp.nonzero(kept_h, size=MAX_HUMAN, fill_value=n)
    (o_pos,) = jnp.nonzero(kept_o, size=MAX_OBJECT, fill_value=n)
    pos_part = jnp.concatenate([h_pos, o_pos])
    c1 = order_part[jnp.clip(nhtot - 1, 0, n - 1)]
    c2 = order_part[n - 1]
    s1 = scores[c1]
    s2 = scores[c2]
    fb = jnp.where(nhtot == 0, c2,
                   jnp.where(nhtot == n, c1,
                             jnp.where(s1 < s2, c1,
                                       jnp.where(s2 < s1, c2,
                                                 jnp.maximum(c1, c2)))))
    sel = jnp.where(pos_part >= n, fb,
                    order_part[jnp.minimum(pos_part, n - 1)])
    b_s = boxes[sel]
    s_s = scores[sel]
    l_s = labels[sel]
    nh = MAX_HUMAN

    no = MAX_OBJECT
    hi = jnp.arange(nh * no) // no
    oi = jnp.arange(nh * no) % no
    prior = (s_s[:nh][hi] * s_s[nh:][oi])[:, None]

    pred = pl.pallas_call(
        _head_kernel,
        out_shape=jax.ShapeDtypeStruct((nh * no, NUM_CLASSES), jnp.float32),
    )(features.reshape(C_FEAT, 64 * 64), b_s, prior,
      W1, b1.reshape(1, -1), W2, b2.reshape(1, -1))

    P = nh * no
    K = NUM_CLASSES
    x = (jnp.arange(P * K) // K).astype(jnp.int32)
    y = (jnp.arange(P * K) % K).astype(jnp.int32)
    pred_scores = pred.reshape(-1)
    boxes_h = b_s[:nh][hi]
    boxes_o = b_s[nh:][oi]
    return boxes_h, boxes_o, x, y, pred_scores, l_s[nh:][oi]
